# Initial kernel scaffold; baseline (speedup 1.0000x reference)
import jax
import jax.numpy as jnp
from jax.experimental import pallas as pl


def kernel(image, stub_w, mean, std, neck1_w, neck1_scale, neck1_bias, neck2_w, neck2_scale, neck2_bias, h1_c1, h1_bn1_scale, h1_bn1_bias, h1_c2, h1_bn2_scale, h1_bn2_bias, h1_w5, h1_b5):
    raise NotImplementedError("write your pallas kernel here")


# speedup vs baseline: 1.0611x; 1.0611x over previous
import functools

import numpy as np
import jax
import jax.numpy as jnp
from jax.experimental import pallas as pl
from jax.experimental.pallas import tpu as pltpu

_CLS = 21


def _whole(*args, nd):
    return (0,) * nd


def _body(x_ref, mke_ref,
          w1_ref, s1_ref, b1_ref,
          w2_ref, s2_ref, b2_ref,
          w3_ref, s3_ref, b3_ref,
          w4_ref, s4_ref, b4_ref,
          wc_ref, bc_ref,
          out_ref, act0, act1,
          *, halo, pitch, rows, tm):
    nt = rows // tm
    gz = halo + pitch
    mbuf = rows + 2 * halo
    shifts = [dy * pitch + dx for dy in (-1, 0, 1) for dx in (-1, 0, 1)]

    def clear_guards(buf):
        z = jnp.zeros((gz, 768), buf.dtype)
        buf[:gz, :] = z
        buf[mbuf - gz:, :] = z

    def store3(dst, y, r0):
        yb = y.astype(dst.dtype)
        dst[halo + r0 + pitch:halo + r0 + pitch + tm, 0:256] = yb
        dst[halo + r0:halo + r0 + tm, 256:512] = yb
        dst[halo + r0 - pitch:halo + r0 - pitch + tm, 512:768] = yb

    clear_guards(act0)
    s1 = s1_ref[...]
    b1 = b1_ref[...]
    for i in range(nt):
        r0 = i * tm
        acc = jnp.zeros((tm, 256), jnp.float32)
        for t, dd in enumerate(shifts):
            a = halo + r0 + dd
            acc += jnp.dot(x_ref[a:a + tm, :], w1_ref[t],
                           preferred_element_type=jnp.float32)
        mk = mke_ref[halo + r0:halo + r0 + tm, :]
        store3(act0, jnp.maximum(acc * s1 + b1, 0.0) * mk, r0)

    def layer(src, wref, sref, bref, dst, last=False):
        sc = sref[...]
        bi = bref[...]
        if not last:
            clear_guards(dst)
        for i in range(nt):
            r0 = i * tm
            acc = jnp.zeros((tm, 256), jnp.float32)
            for j, dx in enumerate((-1, 0, 1)):
                a = halo + r0 + dx
                acc += jnp.dot(src[a:a + tm, :], wref[j],
                               preferred_element_type=jnp.float32)
            mk = mke_ref[halo + r0:halo + r0 + tm, :]
            y = jnp.maximum(acc * sc + bi, 0.0) * mk
            if last:
                out_ref[r0:r0 + tm, :] = (
                    jnp.dot(y.astype(jnp.bfloat16), wc_ref[...],
                            preferred_element_type=jnp.float32) + bc_ref[...])
            else:
                store3(dst, y, r0)

    layer(act0, w2_ref, s2_ref, b2_ref, act1)
    layer(act1, w3_ref, s3_ref, b3_ref, act0)
    layer(act0, w4_ref, s4_ref, b4_ref, None, last=True)


def _stack_ky(w):
    return jnp.transpose(w.reshape(3, 3, 256, 256), (1, 0, 2, 3)).reshape(
        3, 768, 256)


def kernel(image, stub_w, mean, std, neck1_w, neck1_scale, neck1_bias,
           neck2_w, neck2_scale, neck2_bias, h1_c1, h1_bn1_scale, h1_bn1_bias,
           h1_c2, h1_bn2_scale, h1_bn2_bias, h1_w5, h1_b5):
    N, C, H, W = image.shape
    s = 8
    Hf, Wf = H // s, W // s
    Hp = Hf + 2
    pitch = ((Wf + 2 + 7) // 8) * 8
    rows = Hp * pitch
    halo = pitch + 8
    mbuf = rows + 2 * halo
    cpad = h1_w5.shape[1]
    Cin = neck1_w.shape[1]
    tm = 8
    for t in range(480, 7, -8):
        if rows % t == 0:
            tm = t
            break

    pooled = image.reshape(N, C, Hf, s, Wf, s).mean(axis=(3, 5))
    w = stub_w[0] / std[0][:, None]
    b = -(mean[0] / std[0]) @ stub_w[0]
    feat = jnp.einsum('nchw,cd->nhwd', pooled, w,
                      precision=jax.lax.Precision.HIGHEST) + b
    xp = jnp.pad(feat.astype(jnp.bfloat16),
                 ((0, 0), (1, Hp - 1 - Hf), (1, pitch - 1 - Wf), (0, 0)))
    x_flat = jnp.pad(xp.reshape(N, rows, Cin), ((0, 0), (halo, halo), (0, 0)))

    mnp = np.zeros((Hp, pitch), np.float32)
    mnp[1:1 + Hf, 1:1 + Wf] = 1.0
    mke = jnp.asarray(
        np.pad(mnp.reshape(rows), (halo, halo)).reshape(mbuf, 1))

    weights = (neck1_w, neck1_scale, neck1_bias,
               _stack_ky(neck2_w), neck2_scale, neck2_bias,
               _stack_ky(h1_c1), h1_bn1_scale, h1_bn1_bias,
               _stack_ky(h1_c2), h1_bn2_scale, h1_bn2_bias,
               h1_w5, h1_b5)
    wspecs = [pl.BlockSpec(w.shape, functools.partial(_whole, nd=w.ndim))
              for w in weights]

    out = pl.pallas_call(
        functools.partial(_body, halo=halo, pitch=pitch, rows=rows, tm=tm),
        out_shape=jax.ShapeDtypeStruct((N, rows, cpad), jnp.float32),
        grid_spec=pltpu.PrefetchScalarGridSpec(
            num_scalar_prefetch=0,
            grid=(N,),
            in_specs=[pl.BlockSpec((None, mbuf, Cin), lambda n: (n, 0, 0)),
                      pl.BlockSpec((mbuf, 1), lambda n: (0, 0))] + wspecs,
            out_specs=pl.BlockSpec((None, rows, cpad), lambda n: (n, 0, 0)),
            scratch_shapes=[pltpu.VMEM((mbuf, 768), jnp.bfloat16),
                            pltpu.VMEM((mbuf, 768), jnp.bfloat16)]),
        compiler_params=pltpu.CompilerParams(
            dimension_semantics=("parallel",),
            vmem_limit_bytes=40 * 1024 * 1024),
    )(x_flat, mke, *weights)

    pred = (out.reshape(N, Hp, pitch, cpad)
            [:, 1:1 + Hf, 1:1 + Wf, :_CLS])
    return {'pred': jnp.transpose(pred, (0, 3, 1, 2))}
